# Initial kernel scaffold; baseline (speedup 1.0000x reference)
import functools

import jax
import jax.numpy as jnp
from jax import lax
from jax.experimental import pallas as pl
from jax.experimental.pallas import tpu as pltpu
from jax.experimental.pallas import tpu_sc as plsc

VOCAB = 100000
HIDDEN = 1024
MAXPOS = 2048
TYPES = 2
EPS = 1e-12
BATCH = 4
SEQ = 2048

N_TOKENS = BATCH * SEQ

NC = 2
NS = 16
NW = NC * NS

TOK_PER_W = N_TOKENS // NW
CHUNK = 64
N_CHUNKS = TOK_PER_W // CHUNK


def _sc_gather(ids_flat, token_table):
    mesh = plsc.VectorSubcoreMesh(core_axis_name="c", subcore_axis_name="s")

    @functools.partial(
        pl.kernel,
        mesh=mesh,
        out_type=jax.ShapeDtypeStruct((N_TOKENS, HIDDEN), jnp.float32),
        scratch_types=[
            pltpu.VMEM((CHUNK,), jnp.int32),
            pltpu.VMEM((CHUNK, HIDDEN), jnp.float32),
            pltpu.SemaphoreType.DMA,
        ],
    )
    def k(table_hbm, ids_hbm, out_hbm, idx_v, rows_v, sem):
        wid = lax.axis_index("s") * NC + lax.axis_index("c")
        base = wid * TOK_PER_W
        for c in range(N_CHUNKS):
            off = base + c * CHUNK
            pltpu.sync_copy(ids_hbm.at[pl.ds(off, CHUNK)], idx_v)
            pltpu.async_copy(table_hbm.at[idx_v], rows_v, sem).wait()
            pltpu.sync_copy(rows_v, out_hbm.at[pl.ds(off, CHUNK)])

    return k(token_table, ids_flat)


ROWS_BLK = 128
N_BLOCKS = N_TOKENS // ROWS_BLK
POS_BLOCKS = SEQ // ROWS_BLK


def _tc_epilogue_body(tid_ref, tok_ref, pos_ref, typ_ref, gamma_ref, beta_ref,
                      out_ref):
    x = tok_ref[...] + pos_ref[...]
    tid = tid_ref[0, 0, :]
    te = jnp.where((tid == 0)[:, None], typ_ref[0:1, :], typ_ref[1:2, :])
    x = x + te
    mean = jnp.mean(x, axis=-1, keepdims=True)
    xc = x - mean
    var = jnp.mean(xc * xc, axis=-1, keepdims=True)
    y = xc * lax.rsqrt(var + EPS)
    out_ref[...] = y * gamma_ref[...] + beta_ref[...]


def _tc_epilogue(tok_rows, tids_flat, pos_table, type_table, gamma, beta):
    tid3 = tids_flat.reshape(N_BLOCKS, 1, ROWS_BLK)
    return pl.pallas_call(
        _tc_epilogue_body,
        grid=(N_BLOCKS,),
        in_specs=[
            pl.BlockSpec((1, 1, ROWS_BLK), lambda i: (i, 0, 0)),
            pl.BlockSpec((ROWS_BLK, HIDDEN), lambda i: (i, 0)),
            pl.BlockSpec((ROWS_BLK, HIDDEN), lambda i: (i % POS_BLOCKS, 0)),
            pl.BlockSpec((TYPES, HIDDEN), lambda i: (0, 0)),
            pl.BlockSpec((1, HIDDEN), lambda i: (0, 0)),
            pl.BlockSpec((1, HIDDEN), lambda i: (0, 0)),
        ],
        out_specs=pl.BlockSpec((ROWS_BLK, HIDDEN), lambda i: (i, 0)),
        out_shape=jax.ShapeDtypeStruct((N_TOKENS, HIDDEN), jnp.float32),
    )(tid3, tok_rows, pos_table, type_table,
      gamma.reshape(1, HIDDEN), beta.reshape(1, HIDDEN))


def kernel(input_ids, token_type_ids, token_table, pos_table, type_table,
           gamma, beta):
    ids_flat = input_ids.reshape(-1).astype(jnp.int32)
    tids_flat = token_type_ids.reshape(-1).astype(jnp.int32)
    tok_rows = _sc_gather(ids_flat, token_table)
    out = _tc_epilogue(tok_rows, tids_flat, pos_table, type_table, gamma, beta)
    return out.reshape(BATCH, SEQ, HIDDEN)


# speedup vs baseline: 1.2774x; 1.2774x over previous
import functools

import jax
import jax.numpy as jnp
from jax import lax
from jax.experimental import pallas as pl
from jax.experimental.pallas import tpu as pltpu
from jax.experimental.pallas import tpu_sc as plsc

VOCAB = 100000
HIDDEN = 1024
MAXPOS = 2048
TYPES = 2
EPS = 1e-12
BATCH = 4
SEQ = 2048

N_TOKENS = BATCH * SEQ

NC = 2
NS = 16
NW = NC * NS

TOK_PER_W = N_TOKENS // NW
CHUNK = 64
N_CHUNKS = TOK_PER_W // CHUNK


def _sc_gather(ids_flat, token_table):
    mesh = plsc.VectorSubcoreMesh(core_axis_name="c", subcore_axis_name="s")

    @functools.partial(
        pl.kernel,
        mesh=mesh,
        out_type=jax.ShapeDtypeStruct((N_TOKENS, HIDDEN), jnp.float32),
        scratch_types=[
            pltpu.VMEM((CHUNK,), jnp.int32),
            pltpu.VMEM((CHUNK, HIDDEN), jnp.float32),
            pltpu.SemaphoreType.DMA,
        ],
    )
    def k(table_hbm, ids_hbm, out_hbm, idx_v, rows_v, sem):
        wid = lax.axis_index("s") * NC + lax.axis_index("c")
        base = wid * TOK_PER_W
        for c in range(N_CHUNKS):
            off = base + c * CHUNK
            pltpu.sync_copy(ids_hbm.at[pl.ds(off, CHUNK)], idx_v)
            pltpu.async_copy(table_hbm.at[idx_v], rows_v, sem).wait()
            pltpu.sync_copy(rows_v, out_hbm.at[pl.ds(off, CHUNK)])

    return k(token_table, ids_flat)


ROWS_BLK = 128
N_BLOCKS = N_TOKENS // ROWS_BLK
POS_BLOCKS = SEQ // ROWS_BLK


def _tc_epilogue_body(tid_ref, tok_ref, pos_ref, typ_ref, gamma_ref, beta_ref,
                      out_ref):
    x = tok_ref[...] + pos_ref[...]
    te = jnp.where(tid_ref[...] == 0, typ_ref[0:1, :], typ_ref[1:2, :])
    x = x + te
    mean = jnp.mean(x, axis=-1, keepdims=True)
    xc = x - mean
    var = jnp.mean(xc * xc, axis=-1, keepdims=True)
    y = xc * lax.rsqrt(var + EPS)
    out_ref[...] = y * gamma_ref[...] + beta_ref[...]


def _tc_epilogue(tok_rows, tids_flat, pos_table, type_table, gamma, beta):
    tid_col = tids_flat.reshape(N_TOKENS, 1)
    return pl.pallas_call(
        _tc_epilogue_body,
        grid=(N_BLOCKS,),
        in_specs=[
            pl.BlockSpec((ROWS_BLK, 1), lambda i: (i, 0)),
            pl.BlockSpec((ROWS_BLK, HIDDEN), lambda i: (i, 0)),
            pl.BlockSpec((ROWS_BLK, HIDDEN), lambda i: (i % POS_BLOCKS, 0)),
            pl.BlockSpec((TYPES, HIDDEN), lambda i: (0, 0)),
            pl.BlockSpec((1, HIDDEN), lambda i: (0, 0)),
            pl.BlockSpec((1, HIDDEN), lambda i: (0, 0)),
        ],
        out_specs=pl.BlockSpec((ROWS_BLK, HIDDEN), lambda i: (i, 0)),
        out_shape=jax.ShapeDtypeStruct((N_TOKENS, HIDDEN), jnp.float32),
    )(tid_col, tok_rows, pos_table, type_table,
      gamma.reshape(1, HIDDEN), beta.reshape(1, HIDDEN))


def kernel(input_ids, token_type_ids, token_table, pos_table, type_table,
           gamma, beta):
    ids_flat = input_ids.reshape(-1).astype(jnp.int32)
    tids_flat = token_type_ids.reshape(-1).astype(jnp.int32)
    tok_rows = _sc_gather(ids_flat, token_table)
    out = _tc_epilogue(tok_rows, tids_flat, pos_table, type_table, gamma, beta)
    return out.reshape(BATCH, SEQ, HIDDEN)


# speedup vs baseline: 1.3638x; 1.0677x over previous
import functools

import jax
import jax.numpy as jnp
from jax import lax
from jax.experimental import pallas as pl
from jax.experimental.pallas import tpu as pltpu
from jax.experimental.pallas import tpu_sc as plsc

VOCAB = 100000
HIDDEN = 1024
MAXPOS = 2048
TYPES = 2
EPS = 1e-12
BATCH = 4
SEQ = 2048

N_TOKENS = BATCH * SEQ

NC = 2
NS = 16
NW = NC * NS

N_SLICES = 4
SLICE = N_TOKENS // N_SLICES
TOK_PER_W = SLICE // NW
CHUNK = 64
N_CHUNKS = max(TOK_PER_W // CHUNK, 1)
CHUNK = TOK_PER_W // N_CHUNKS


def _sc_gather(ids_flat, token_table):
    mesh = plsc.VectorSubcoreMesh(core_axis_name="c", subcore_axis_name="s")

    @functools.partial(
        pl.kernel,
        mesh=mesh,
        out_type=jax.ShapeDtypeStruct((SLICE, HIDDEN), jnp.float32),
        scratch_types=[
            pltpu.VMEM((CHUNK,), jnp.int32),
            pltpu.VMEM((CHUNK, HIDDEN), jnp.float32),
            pltpu.SemaphoreType.DMA,
        ],
    )
    def k(table_hbm, ids_hbm, out_hbm, idx_v, rows_v, sem):
        wid = lax.axis_index("s") * NC + lax.axis_index("c")
        base = wid * TOK_PER_W
        for c in range(N_CHUNKS):
            off = base + c * CHUNK
            pltpu.sync_copy(ids_hbm.at[pl.ds(off, CHUNK)], idx_v)
            pltpu.async_copy(table_hbm.at[idx_v], rows_v, sem).wait()
            pltpu.sync_copy(rows_v, out_hbm.at[pl.ds(off, CHUNK)])

    return k(token_table, ids_flat)


ROWS_BLK = 128
BLK_PER_SLICE = SLICE // ROWS_BLK
POS_BLOCKS = SEQ // ROWS_BLK


def _tc_epilogue_body(tid_ref, tok_ref, pos_ref, typ_ref, gamma_ref, beta_ref,
                      out_ref):
    x = tok_ref[...] + pos_ref[...]
    te = jnp.where(tid_ref[...] == 0, typ_ref[0:1, :], typ_ref[1:2, :])
    x = x + te
    mean = jnp.mean(x, axis=-1, keepdims=True)
    xc = x - mean
    var = jnp.mean(xc * xc, axis=-1, keepdims=True)
    y = xc * lax.rsqrt(var + EPS)
    out_ref[...] = y * gamma_ref[...] + beta_ref[...]


def _tc_epilogue_slice(s, acc, tok_rows, tid_col, pos_table, type_table,
                       gamma2d, beta2d):
    blk0 = s * BLK_PER_SLICE

    def body(tid_ref, tok_ref, pos_ref, typ_ref, gamma_ref, beta_ref,
             *rest):
        out_ref = rest[-1]
        _tc_epilogue_body(tid_ref, tok_ref, pos_ref, typ_ref, gamma_ref,
                          beta_ref, out_ref)

    in_specs = [
        pl.BlockSpec((ROWS_BLK, 1), lambda i: (i, 0)),
        pl.BlockSpec((ROWS_BLK, HIDDEN), lambda i: (i, 0)),
        pl.BlockSpec((ROWS_BLK, HIDDEN),
                     lambda i: ((blk0 + i) % POS_BLOCKS, 0)),
        pl.BlockSpec((TYPES, HIDDEN), lambda i: (0, 0)),
        pl.BlockSpec((1, HIDDEN), lambda i: (0, 0)),
        pl.BlockSpec((1, HIDDEN), lambda i: (0, 0)),
    ]
    args = [tid_col, tok_rows, pos_table, type_table, gamma2d, beta2d]
    io_aliases = {}
    if acc is not None:
        in_specs.append(pl.BlockSpec(memory_space=pl.ANY))
        args.append(acc)
        io_aliases = {6: 0}
    return pl.pallas_call(
        body,
        grid=(BLK_PER_SLICE,),
        in_specs=in_specs,
        out_specs=pl.BlockSpec((ROWS_BLK, HIDDEN), lambda i: (blk0 + i, 0)),
        out_shape=jax.ShapeDtypeStruct((N_TOKENS, HIDDEN), jnp.float32),
        input_output_aliases=io_aliases,
    )(*args)


def kernel(input_ids, token_type_ids, token_table, pos_table, type_table,
           gamma, beta):
    ids_flat = input_ids.reshape(-1).astype(jnp.int32)
    tid_col = token_type_ids.reshape(N_TOKENS, 1).astype(jnp.int32)
    gamma2d = gamma.reshape(1, HIDDEN)
    beta2d = beta.reshape(1, HIDDEN)

    gathered = [
        _sc_gather(lax.slice(ids_flat, (s * SLICE,), ((s + 1) * SLICE,)),
                   token_table)
        for s in range(N_SLICES)
    ]
    acc = None
    for s in range(N_SLICES):
        tid_slice = lax.slice(tid_col, (s * SLICE, 0), ((s + 1) * SLICE, 1))
        acc = _tc_epilogue_slice(s, acc, gathered[s], tid_slice, pos_table,
                                 type_table, gamma2d, beta2d)
    return acc.reshape(BATCH, SEQ, HIDDEN)


# speedup vs baseline: 1.4040x; 1.0295x over previous
import functools

import jax
import jax.numpy as jnp
from jax import lax
from jax.experimental import pallas as pl
from jax.experimental.pallas import tpu as pltpu
from jax.experimental.pallas import tpu_sc as plsc

VOCAB = 100000
HIDDEN = 1024
MAXPOS = 2048
TYPES = 2
EPS = 1e-12
BATCH = 4
SEQ = 2048

N_TOKENS = BATCH * SEQ

NC = 2
NS = 16
NW = NC * NS

N_SLICES = 4
SEQ_SLICE = SEQ // N_SLICES
SLICE = BATCH * SEQ_SLICE
TOK_PER_W = SLICE // NW
CHUNK = 32
N_CHUNKS = TOK_PER_W // CHUNK


def _sc_gather(ids_flat, token_table):
    mesh = plsc.VectorSubcoreMesh(core_axis_name="c", subcore_axis_name="s")

    @functools.partial(
        pl.kernel,
        mesh=mesh,
        out_type=jax.ShapeDtypeStruct((SLICE, HIDDEN), jnp.float32),
        scratch_types=[
            pltpu.VMEM((TOK_PER_W,), jnp.int32),
            pltpu.VMEM((N_CHUNKS, CHUNK, HIDDEN), jnp.float32),
            pltpu.SemaphoreType.DMA((N_CHUNKS,)),
            pltpu.SemaphoreType.DMA((N_CHUNKS,)),
        ],
    )
    def k(table_hbm, ids_hbm, out_hbm, idx_v, rows_v, gsem, osem):
        wid = lax.axis_index("s") * NC + lax.axis_index("c")
        base = wid * TOK_PER_W
        pltpu.sync_copy(ids_hbm.at[pl.ds(base, TOK_PER_W)], idx_v)
        gathers = []
        for c in range(N_CHUNKS):
            g = pltpu.async_copy(
                table_hbm.at[idx_v.at[pl.ds(c * CHUNK, CHUNK)]],
                rows_v.at[c], gsem.at[c])
            gathers.append(g)
        outs = []
        for c in range(N_CHUNKS):
            gathers[c].wait()
            o = pltpu.async_copy(
                rows_v.at[c], out_hbm.at[pl.ds(base + c * CHUNK, CHUNK)],
                osem.at[c])
            outs.append(o)
        for o in outs:
            o.wait()

    return k(token_table, ids_flat)


ROWS_BLK = 128
BLK_PER_SEQ_SLICE = SEQ_SLICE // ROWS_BLK
POS_BLOCKS = SEQ // ROWS_BLK


def _tc_epilogue_body(tid_ref, tok_ref, pos_ref, typ_ref, gamma_ref, beta_ref,
                      *rest):
    out_ref = rest[-1]
    x = tok_ref[...] + pos_ref[...]
    te = jnp.where(tid_ref[...] == 0, typ_ref[0:1, :], typ_ref[1:2, :])
    x = x + te
    mean = jnp.mean(x, axis=-1, keepdims=True)
    xc = x - mean
    var = jnp.mean(xc * xc, axis=-1, keepdims=True)
    y = xc * lax.rsqrt(var + EPS)
    out_ref[...] = y * gamma_ref[...] + beta_ref[...]


def _tc_epilogue_slice(s, acc, tok_rows, tid_col, pos_table, type_table,
                       gamma2d, beta2d):
    pos_blk0 = s * BLK_PER_SEQ_SLICE

    in_specs = [
        pl.BlockSpec((ROWS_BLK, 1), lambda i, b: (b * BLK_PER_SEQ_SLICE + i, 0)),
        pl.BlockSpec((ROWS_BLK, HIDDEN),
                     lambda i, b: (b * BLK_PER_SEQ_SLICE + i, 0)),
        pl.BlockSpec((ROWS_BLK, HIDDEN), lambda i, b: (pos_blk0 + i, 0)),
        pl.BlockSpec((TYPES, HIDDEN), lambda i, b: (0, 0)),
        pl.BlockSpec((1, HIDDEN), lambda i, b: (0, 0)),
        pl.BlockSpec((1, HIDDEN), lambda i, b: (0, 0)),
    ]
    args = [tid_col, tok_rows, pos_table, type_table, gamma2d, beta2d]
    io_aliases = {}
    if acc is not None:
        in_specs.append(pl.BlockSpec(memory_space=pl.ANY))
        args.append(acc)
        io_aliases = {6: 0}
    return pl.pallas_call(
        _tc_epilogue_body,
        grid=(BLK_PER_SEQ_SLICE, BATCH),
        in_specs=in_specs,
        out_specs=pl.BlockSpec(
            (ROWS_BLK, HIDDEN),
            lambda i, b: (b * POS_BLOCKS + pos_blk0 + i, 0)),
        out_shape=jax.ShapeDtypeStruct((N_TOKENS, HIDDEN), jnp.float32),
        input_output_aliases=io_aliases,
    )(*args)


def kernel(input_ids, token_type_ids, token_table, pos_table, type_table,
           gamma, beta):
    ids2d = input_ids.astype(jnp.int32)
    tids2d = token_type_ids.astype(jnp.int32)
    gamma2d = gamma.reshape(1, HIDDEN)
    beta2d = beta.reshape(1, HIDDEN)

    gathered = []
    tid_slices = []
    for s in range(N_SLICES):
        lo, hi = s * SEQ_SLICE, (s + 1) * SEQ_SLICE
        ids_slice = lax.slice(ids2d, (0, lo), (BATCH, hi)).reshape(-1)
        tid_slices.append(
            lax.slice(tids2d, (0, lo), (BATCH, hi)).reshape(SLICE, 1))
        gathered.append(_sc_gather(ids_slice, token_table))

    acc = None
    for s in range(N_SLICES):
        acc = _tc_epilogue_slice(s, acc, gathered[s], tid_slices[s],
                                 pos_table, type_table, gamma2d, beta2d)
    return acc.reshape(BATCH, SEQ, HIDDEN)


# speedup vs baseline: 1.7045x; 1.2140x over previous
import functools

import jax
import jax.numpy as jnp
from jax import lax
from jax.experimental import pallas as pl
from jax.experimental.pallas import tpu as pltpu
from jax.experimental.pallas import tpu_sc as plsc

VOCAB = 100000
HIDDEN = 1024
MAXPOS = 2048
TYPES = 2
EPS = 1e-12
BATCH = 4
SEQ = 2048

N_TOKENS = BATCH * SEQ

NC = 2
NS = 16
NW = NC * NS

N_SLICES = 4
SEQ_SLICE = SEQ // N_SLICES
SLICE = BATCH * SEQ_SLICE
TOK_PER_W = SLICE // NW
CHUNK = 32
N_CHUNKS = TOK_PER_W // CHUNK


def _sc_gather(s, ids_flat, token_table):
    mesh = plsc.VectorSubcoreMesh(core_axis_name="c", subcore_axis_name="s")

    @functools.partial(
        pl.kernel,
        mesh=mesh,
        out_type=jax.ShapeDtypeStruct((SLICE, HIDDEN), jnp.float32),
        scratch_types=[
            pltpu.VMEM((TOK_PER_W,), jnp.int32),
            pltpu.VMEM((N_CHUNKS, CHUNK, HIDDEN), jnp.float32),
            pltpu.SemaphoreType.DMA((N_CHUNKS,)),
            pltpu.SemaphoreType.DMA((N_CHUNKS,)),
        ],
    )
    def k(table_hbm, ids_hbm, out_hbm, idx_v, rows_v, gsem, osem):
        wid = lax.axis_index("s") * NC + lax.axis_index("c")
        r = wid * TOK_PER_W
        b = r // SEQ_SLICE
        p0 = r % SEQ_SLICE
        gbase = b * SEQ + s * SEQ_SLICE + p0
        pltpu.sync_copy(ids_hbm.at[pl.ds(gbase, TOK_PER_W)], idx_v)
        gathers = []
        for c in range(N_CHUNKS):
            g = pltpu.async_copy(
                table_hbm.at[idx_v.at[pl.ds(c * CHUNK, CHUNK)]],
                rows_v.at[c], gsem.at[c])
            gathers.append(g)
        outs = []
        for c in range(N_CHUNKS):
            gathers[c].wait()
            o = pltpu.async_copy(
                rows_v.at[c], out_hbm.at[pl.ds(r + c * CHUNK, CHUNK)],
                osem.at[c])
            outs.append(o)
        for o in outs:
            o.wait()

    return k(token_table, ids_flat)


ROWS_BLK = 256
BLK_PER_SEQ_SLICE = SEQ_SLICE // ROWS_BLK
SEQ_BLOCKS = SEQ // ROWS_BLK


def _tc_epilogue_body(tid_ref, tok_ref, pos_ref, typ_ref, gamma_ref, beta_ref,
                      *rest):
    out_ref = rest[-1]
    x = tok_ref[...] + pos_ref[...]
    te = jnp.where(tid_ref[...] == 0, typ_ref[0:1, :], typ_ref[1:2, :])
    x = x + te
    mean = jnp.mean(x, axis=-1, keepdims=True)
    xc = x - mean
    var = jnp.mean(xc * xc, axis=-1, keepdims=True)
    y = xc * lax.rsqrt(var + EPS)
    out_ref[...] = y * gamma_ref[...] + beta_ref[...]


def _tc_epilogue_slice(s, acc, tok_rows, tid_col, pos_table, type_table,
                       gamma2d, beta2d):
    blk0 = s * BLK_PER_SEQ_SLICE

    in_specs = [
        pl.BlockSpec((ROWS_BLK, 1),
                     lambda i, b: (b * SEQ_BLOCKS + blk0 + i, 0)),
        pl.BlockSpec((ROWS_BLK, HIDDEN),
                     lambda i, b: (b * BLK_PER_SEQ_SLICE + i, 0)),
        pl.BlockSpec((ROWS_BLK, HIDDEN), lambda i, b: (blk0 + i, 0)),
        pl.BlockSpec((TYPES, HIDDEN), lambda i, b: (0, 0)),
        pl.BlockSpec((1, HIDDEN), lambda i, b: (0, 0)),
        pl.BlockSpec((1, HIDDEN), lambda i, b: (0, 0)),
    ]
    args = [tid_col, tok_rows, pos_table, type_table, gamma2d, beta2d]
    io_aliases = {}
    if acc is not None:
        in_specs.append(pl.BlockSpec(memory_space=pl.ANY))
        args.append(acc)
        io_aliases = {6: 0}
    return pl.pallas_call(
        _tc_epilogue_body,
        grid=(BLK_PER_SEQ_SLICE, BATCH),
        in_specs=in_specs,
        out_specs=pl.BlockSpec(
            (ROWS_BLK, HIDDEN),
            lambda i, b: (b * SEQ_BLOCKS + blk0 + i, 0)),
        out_shape=jax.ShapeDtypeStruct((N_TOKENS, HIDDEN), jnp.float32),
        input_output_aliases=io_aliases,
    )(*args)


def kernel(input_ids, token_type_ids, token_table, pos_table, type_table,
           gamma, beta):
    ids_flat = input_ids.reshape(-1).astype(jnp.int32)
    tid_col = token_type_ids.reshape(N_TOKENS, 1).astype(jnp.int32)
    gamma2d = gamma.reshape(1, HIDDEN)
    beta2d = beta.reshape(1, HIDDEN)

    gathered = [_sc_gather(s, ids_flat, token_table)
                for s in range(N_SLICES)]
    acc = None
    for s in range(N_SLICES):
        acc = _tc_epilogue_slice(s, acc, gathered[s], tid_col,
                                 pos_table, type_table, gamma2d, beta2d)
    return acc.reshape(BATCH, SEQ, HIDDEN)


# speedup vs baseline: 1.7879x; 1.0489x over previous
import functools

import jax
import jax.numpy as jnp
from jax import lax
from jax.experimental import pallas as pl
from jax.experimental.pallas import tpu as pltpu
from jax.experimental.pallas import tpu_sc as plsc

VOCAB = 100000
HIDDEN = 1024
MAXPOS = 2048
TYPES = 2
EPS = 1e-12
BATCH = 4
SEQ = 2048

N_TOKENS = BATCH * SEQ

NC = 2
NS = 16
NW = NC * NS

N_SLICES = 4
SEQ_SLICE = SEQ // N_SLICES
SLICE = BATCH * SEQ_SLICE
TOK_PER_W = SLICE // NW
CHUNK = 32
N_CHUNKS = TOK_PER_W // CHUNK


def _sc_gather(s, ids_flat, token_table):
    mesh = plsc.VectorSubcoreMesh(core_axis_name="c", subcore_axis_name="s")

    @functools.partial(
        pl.kernel,
        mesh=mesh,
        out_type=jax.ShapeDtypeStruct((SLICE, HIDDEN), jnp.float32),
        scratch_types=[
            pltpu.VMEM((TOK_PER_W,), jnp.int32),
            pltpu.VMEM((N_CHUNKS, CHUNK, HIDDEN), jnp.float32),
            pltpu.SemaphoreType.DMA((N_CHUNKS,)),
            pltpu.SemaphoreType.DMA((N_CHUNKS,)),
        ],
    )
    def k(table_hbm, ids_hbm, out_hbm, idx_v, rows_v, gsem, osem):
        wid = lax.axis_index("s") * NC + lax.axis_index("c")
        r = wid * TOK_PER_W
        b = r // SEQ_SLICE
        p0 = r % SEQ_SLICE
        gbase = b * SEQ + s * SEQ_SLICE + p0
        pltpu.sync_copy(ids_hbm.at[pl.ds(gbase, TOK_PER_W)], idx_v)
        gathers = []
        for c in range(N_CHUNKS):
            g = pltpu.async_copy(
                table_hbm.at[idx_v.at[pl.ds(c * CHUNK, CHUNK)]],
                rows_v.at[c], gsem.at[c])
            gathers.append(g)
        outs = []
        for c in range(N_CHUNKS):
            gathers[c].wait()
            o = pltpu.async_copy(
                rows_v.at[c], out_hbm.at[pl.ds(r + c * CHUNK, CHUNK)],
                osem.at[c])
            outs.append(o)
        for o in outs:
            o.wait()

    return k(token_table, ids_flat)


ROWS_BLK = 512
BLK_PER_SEQ_SLICE = SEQ_SLICE // ROWS_BLK
SEQ_BLOCKS = SEQ // ROWS_BLK


def _tc_epilogue_body(tid_ref, tok_ref, pos_ref, typ_ref, gamma_ref, beta_ref,
                      *rest):
    out_ref = rest[-1]
    x = tok_ref[...] + pos_ref[...]
    te = jnp.where(tid_ref[...] == 0, typ_ref[0:1, :], typ_ref[1:2, :])
    x = x + te
    mean = jnp.mean(x, axis=-1, keepdims=True)
    xc = x - mean
    var = jnp.mean(xc * xc, axis=-1, keepdims=True)
    y = xc * lax.rsqrt(var + EPS)
    out_ref[...] = y * gamma_ref[...] + beta_ref[...]


def _tc_epilogue_slice(s, acc, tok_rows, tid_col, pos_table, type_table,
                       gamma2d, beta2d):
    blk0 = s * BLK_PER_SEQ_SLICE

    in_specs = [
        pl.BlockSpec((ROWS_BLK, 1),
                     lambda i, b: (b * SEQ_BLOCKS + blk0 + i, 0)),
        pl.BlockSpec((ROWS_BLK, HIDDEN),
                     lambda i, b: (b * BLK_PER_SEQ_SLICE + i, 0)),
        pl.BlockSpec((ROWS_BLK, HIDDEN), lambda i, b: (blk0 + i, 0)),
        pl.BlockSpec((TYPES, HIDDEN), lambda i, b: (0, 0)),
        pl.BlockSpec((1, HIDDEN), lambda i, b: (0, 0)),
        pl.BlockSpec((1, HIDDEN), lambda i, b: (0, 0)),
    ]
    args = [tid_col, tok_rows, pos_table, type_table, gamma2d, beta2d]
    io_aliases = {}
    if acc is not None:
        in_specs.append(pl.BlockSpec(memory_space=pl.ANY))
        args.append(acc)
        io_aliases = {6: 0}
    return pl.pallas_call(
        _tc_epilogue_body,
        grid=(BLK_PER_SEQ_SLICE, BATCH),
        in_specs=in_specs,
        out_specs=pl.BlockSpec(
            (ROWS_BLK, HIDDEN),
            lambda i, b: (b * SEQ_BLOCKS + blk0 + i, 0)),
        out_shape=jax.ShapeDtypeStruct((N_TOKENS, HIDDEN), jnp.float32),
        input_output_aliases=io_aliases,
    )(*args)


def kernel(input_ids, token_type_ids, token_table, pos_table, type_table,
           gamma, beta):
    ids_flat = input_ids.reshape(-1).astype(jnp.int32)
    tid_col = token_type_ids.reshape(N_TOKENS, 1).astype(jnp.int32)
    gamma2d = gamma.reshape(1, HIDDEN)
    beta2d = beta.reshape(1, HIDDEN)

    gathered = [_sc_gather(s, ids_flat, token_table)
                for s in range(N_SLICES)]
    acc = None
    for s in range(N_SLICES):
        acc = _tc_epilogue_slice(s, acc, gathered[s], tid_col,
                                 pos_table, type_table, gamma2d, beta2d)
    return acc.reshape(BATCH, SEQ, HIDDEN)
